# Initial kernel scaffold; baseline (speedup 1.0000x reference)
import functools

import jax
import jax.numpy as jnp
from jax import lax
from jax.experimental import pallas as pl
from jax.experimental.pallas import tpu as pltpu
from jax.experimental.pallas import tpu_sc as plsc

SMOOTH = 0.1
CONF = 1.0 - SMOOTH

NC = 2
NS = 16
L = 16
NW = NC * NS

CH = 16


def _make_sc_loss(n_rows, v):
    rows_per_w = n_rows // NW
    eps = SMOOTH / (v - 1)
    import math
    c_const = CONF * math.log(CONF) + (v - 1) * eps * math.log(eps)
    n_full = v // L
    tail = v - n_full * L
    n_cvec = rows_per_w // L
    idx_cap = rows_per_w + CH

    mesh = plsc.VectorSubcoreMesh(
        core_axis_name="c", subcore_axis_name="s",
        num_cores=NC, num_subcores=NS)

    @functools.partial(
        pl.kernel,
        out_type=(
            jax.ShapeDtypeStruct((NW * L,), jnp.float32),
            jax.ShapeDtypeStruct((NW * L,), jnp.float32),
        ),
        mesh=mesh,
        scratch_types=[
            pltpu.VMEM((rows_per_w,), jnp.int32),
            pltpu.VMEM((rows_per_w,), jnp.int32),
            pltpu.VMEM((idx_cap,), jnp.int32),
            pltpu.VMEM((idx_cap,), jnp.int32),
            pltpu.VMEM((CH, v), jnp.float32),
            pltpu.VMEM((L,), jnp.float32),
            pltpu.VMEM((L,), jnp.float32),
            pltpu.SemaphoreType.DMA,
        ],
    )
    def sc_loss(pred_hbm, tgt_hbm, msk_hbm, numer_hbm, cnt_hbm,
                mvec, tvec, idx_v, tgt_c, buf, stage_n, stage_c, sem):
        wid = lax.axis_index("s") * NC + lax.axis_index("c")
        base = wid * rows_per_w
        iota = lax.iota(jnp.int32, L)

        pltpu.sync_copy(msk_hbm.at[pl.ds(base, rows_per_w)], mvec)
        pltpu.sync_copy(tgt_hbm.at[pl.ds(base, rows_per_w)], tvec)

        def zero_body(j, _):
            z = jnp.zeros((L,), jnp.int32)
            idx_v[pl.ds(j * L, L)] = z
            tgt_c[pl.ds(j * L, L)] = z
            return 0

        lax.fori_loop(0, idx_cap // L, zero_body, 0)

        def compact_body(j, n):
            mv = mvec[pl.ds(j * L, L)]
            keep = mv > 0
            rows = base + j * L + iota
            tv = tvec[pl.ds(j * L, L)]
            plsc.store_compressed(idx_v.at[pl.ds(n, L)], rows, mask=keep)
            plsc.store_compressed(tgt_c.at[pl.ds(n, L)], tv, mask=keep)
            return n + jnp.sum(keep.astype(jnp.int32))

        n = lax.fori_loop(0, n_cvec, compact_body, jnp.int32(0))
        n_chunks = (n + (CH - 1)) // CH

        def chunk_body(t, carry):
            a0, a1, a2, a3, gacc = carry
            idx_ref = idx_v.at[pl.ds(t * CH, CH)]
            pltpu.async_copy(pred_hbm.at[idx_ref], buf, sem).wait()

            def row_body(i, accs):
                b0, b1, b2, b3 = accs
                wf = jnp.where(t * CH + i < n, 1.0, 0.0)
                accs = [b0, b1, b2, b3]
                for j in range(n_full):
                    accs[j % 4] = accs[j % 4] + wf * buf[i, pl.ds(j * L, L)]
                if tail:
                    xt = buf[i, pl.ds(v - L, L)]
                    xt = jnp.where(iota >= L - tail, xt, 0.0)
                    accs[n_full % 4] = accs[n_full % 4] + wf * xt
                return tuple(accs)

            a0, a1, a2, a3 = lax.fori_loop(0, CH, row_body, (a0, a1, a2, a3))

            tch = tgt_c[pl.ds(t * CH, CH)]
            g = plsc.load_gather(buf, [iota, tch])
            gacc = gacc + jnp.where(t * CH + iota < n, g, 0.0)
            return a0, a1, a2, a3, gacc

        zv = jnp.zeros((L,), jnp.float32)
        a0, a1, a2, a3, gacc = lax.fori_loop(
            0, n_chunks, chunk_body, (zv, zv, zv, zv, zv))

        sacc = (a0 + a1) + (a2 + a3)
        nf = n.astype(jnp.float32)
        numer = -eps * sacc - (CONF - eps) * gacc
        numer = numer + jnp.where(iota == 0, c_const * nf, 0.0)
        stage_n[...] = numer
        stage_c[...] = jnp.where(iota == 0, nf, 0.0)
        pltpu.sync_copy(stage_n, numer_hbm.at[pl.ds(wid * L, L)])
        pltpu.sync_copy(stage_c, cnt_hbm.at[pl.ds(wid * L, L)])

    return sc_loss


def kernel(prediction, target, mask):
    v = prediction.shape[-1]
    p = prediction.reshape(-1, v)
    t = target.reshape(-1).astype(jnp.int32)
    m = mask.reshape(-1).astype(jnp.int32)
    numer, cnt = _make_sc_loss(p.shape[0], v)(p, t, m)
    return jnp.sum(numer) / jnp.sum(cnt)


# speedup vs baseline: 1.4431x; 1.4431x over previous
import functools
import math

import jax
import jax.numpy as jnp
from jax import lax
from jax.experimental import pallas as pl
from jax.experimental.pallas import tpu as pltpu
from jax.experimental.pallas import tpu_sc as plsc

SMOOTH = 0.1
CONF = 1.0 - SMOOTH

NC = 2
NS = 16
L = 16
NW = NC * NS

CH = 16


def _make_sc_loss(n_rows, v):
    rows = n_rows // NW
    eps = SMOOTH / (v - 1)
    c_const = CONF * math.log(CONF) + (v - 1) * eps * math.log(eps)
    n_full = v // L
    tail = v - n_full * L

    mesh = plsc.VectorSubcoreMesh(
        core_axis_name="c", subcore_axis_name="s",
        num_cores=NC, num_subcores=NS)

    @functools.partial(
        pl.kernel,
        out_type=(
            jax.ShapeDtypeStruct((NW * L,), jnp.float32),
            jax.ShapeDtypeStruct((NW * L,), jnp.float32),
        ),
        mesh=mesh,
        compiler_params=pltpu.CompilerParams(
            needs_layout_passes=False, use_tc_tiling_on_sc=False),
        scratch_types=[
            pltpu.VMEM((rows,), jnp.int32),
            pltpu.VMEM((rows + CH + L,), jnp.int32),
            pltpu.VMEM((rows,), jnp.int32),
            pltpu.VMEM((rows + CH + L,), jnp.int32),
            pltpu.VMEM((CH, v), jnp.float32),
            pltpu.VMEM((L,), jnp.float32),
            pltpu.VMEM((L,), jnp.float32),
            pltpu.SemaphoreType.DMA,
        ],
    )
    def k(pred_hbm, tgt_hbm, msk_hbm, out_hbm, out2_hbm,
          mvec, idx_v, tvec, tgt_c, buf, stage, stage2, sem):
        wid = lax.axis_index("s") * NC + lax.axis_index("c")
        base = wid * rows
        iota = lax.iota(jnp.int32, L)

        def bc(x, dtype):
            return lax.broadcast(jnp.asarray(x, dtype), (L,))

        pltpu.sync_copy(msk_hbm.at[pl.ds(base, rows)], mvec)
        pltpu.sync_copy(tgt_hbm.at[pl.ds(base, rows)], tvec)

        def zero_body(j, _):
            idx_v[pl.ds(j * L, L)] = jnp.zeros((L,), jnp.int32)
            tgt_c[pl.ds(j * L, L)] = jnp.zeros((L,), jnp.int32)
            return 0

        lax.fori_loop(0, (rows + CH + L) // L, zero_body, 0)

        def compact_body(j, nn):
            mv = mvec[pl.ds(j * L, L)]
            keep = mv > 0
            keep_i = jnp.where(keep, jnp.int32(1), jnp.int32(0))
            rowids = bc(base + j * L, jnp.int32) + iota
            csum = plsc.cumsum(keep_i)
            pos = jnp.where(keep, bc(nn, jnp.int32) + csum - 1,
                            (rows + CH) + iota)
            plsc.store_scatter(idx_v, [pos], rowids)
            tv = tvec[pl.ds(j * L, L)]
            plsc.store_scatter(tgt_c, [pos], tv)
            return nn + jnp.sum(keep_i)

        n = lax.fori_loop(0, rows // L, compact_body, jnp.int32(0))

        n_chunks = (n + CH - 1) // CH
        zv = jnp.zeros((L,), jnp.float32)

        def chunk_body(t, carry):
            a0, a1, a2, a3, gacc = carry
            idx_ref = idx_v.at[pl.ds(t * CH, CH)]
            pltpu.async_copy(pred_hbm.at[idx_ref], buf, sem).wait()

            def row_body(i, accs):
                wf = jnp.where(
                    bc(t * CH + i, jnp.int32) < bc(n, jnp.int32), 1.0, 0.0)
                accs = list(accs)
                for j in range(n_full):
                    accs[j % 4] = accs[j % 4] + wf * buf[i, pl.ds(j * L, L)]
                if tail:
                    xt = buf[i, pl.ds(v - L, L)]
                    xt = jnp.where(iota >= L - tail, xt, 0.0)
                    accs[n_full % 4] = accs[n_full % 4] + wf * xt
                return tuple(accs)

            a0, a1, a2, a3 = lax.fori_loop(0, CH, row_body, (a0, a1, a2, a3))
            tch = tgt_c[pl.ds(t * CH, CH)]
            g = plsc.load_gather(buf, [iota, tch])
            valid = bc(t * CH, jnp.int32) + iota < bc(n, jnp.int32)
            gacc = gacc + jnp.where(valid, g, 0.0)
            return (a0, a1, a2, a3, gacc)

        a0, a1, a2, a3, gacc = lax.fori_loop(
            0, n_chunks, chunk_body, (zv, zv, zv, zv, zv))

        sacc = (a0 + a1) + (a2 + a3)
        nfv = bc(n.astype(jnp.float32), jnp.float32)
        lane0 = iota == 0
        numer = -eps * sacc - (CONF - eps) * gacc
        numer = numer + jnp.where(lane0, c_const * nfv, 0.0)
        stage[...] = numer
        pltpu.sync_copy(stage, out_hbm.at[pl.ds(wid * L, L)])
        stage2[...] = jnp.where(lane0, nfv, 0.0)
        pltpu.sync_copy(stage2, out2_hbm.at[pl.ds(wid * L, L)])

    return k


def kernel(prediction, target, mask):
    v = prediction.shape[-1]
    p = prediction.reshape(-1, v)
    t = target.reshape(-1).astype(jnp.int32)
    m = mask.reshape(-1).astype(jnp.int32)
    numer, cnt = _make_sc_loss(p.shape[0], v)(p, t, m)
    return jnp.sum(numer) / jnp.sum(cnt)


# speedup vs baseline: 2.0404x; 1.4139x over previous
import functools
import math

import jax
import jax.numpy as jnp
from jax import lax
from jax.experimental import pallas as pl
from jax.experimental.pallas import tpu as pltpu
from jax.experimental.pallas import tpu_sc as plsc

SMOOTH = 0.1
CONF = 1.0 - SMOOTH
NC, NS, L = 2, 16, 16
NW = NC * NS
CH = 16


def _make_sc_loss(n_rows, v):
    rows = n_rows // NW
    eps = SMOOTH / (v - 1)
    c_const = CONF * math.log(CONF) + (v - 1) * eps * math.log(eps)
    n_full = v // L
    tail = v - n_full * L

    mesh = plsc.VectorSubcoreMesh(
        core_axis_name="c", subcore_axis_name="s",
        num_cores=NC, num_subcores=NS)

    @functools.partial(
        pl.kernel,
        out_type=(
            jax.ShapeDtypeStruct((NW * L,), jnp.float32),
            jax.ShapeDtypeStruct((NW * L,), jnp.float32),
        ),
        mesh=mesh,
        compiler_params=pltpu.CompilerParams(needs_layout_passes=False),
        scratch_types=[
            pltpu.VMEM((rows + L,), jnp.int32),
            pltpu.VMEM((rows + L,), jnp.int32),
            pltpu.VMEM((CH, v), jnp.float32),
            pltpu.VMEM((L,), jnp.float32),
            pltpu.VMEM((L,), jnp.float32),
            pltpu.SemaphoreType.DMA,
        ],
    )
    def k(pred_hbm, tgt_hbm, msk_hbm, out_hbm, out2_hbm,
          mvec, tvec, buf, stage, stage2, sem):
        wid = lax.axis_index("s") * NC + lax.axis_index("c")
        base = wid * rows
        iota = lax.iota(jnp.int32, L)

        def bc(x, dtype):
            return lax.broadcast(jnp.asarray(x, dtype), (L,))

        pltpu.sync_copy(msk_hbm.at[pl.ds(base, rows)],
                        mvec.at[pl.ds(0, rows)])
        pltpu.sync_copy(tgt_hbm.at[pl.ds(base, rows)],
                        tvec.at[pl.ds(0, rows)])

        zv = jnp.zeros((L,), jnp.float32)

        def chunk_body(t, carry):
            a0, a1, a2, a3, gacc, nacc = carry
            pltpu.async_copy(
                pred_hbm.at[pl.ds(base + t * CH, CH)], buf, sem).wait()
            mv16 = mvec[pl.ds(t * CH, L)]
            wf16 = jnp.where(mv16 > 0, 1.0, 0.0)
            tch = tvec[pl.ds(t * CH, L)]
            g = plsc.load_gather(buf, [iota, tch])
            gacc = gacc + wf16 * g
            nacc = nacc + wf16

            def row_body(i, accs):
                m = mvec[pl.ds(t * CH + i, L)][0]
                wfv = bc(m.astype(jnp.float32), jnp.float32)
                accs = list(accs)
                for j in range(n_full):
                    accs[j % 4] = accs[j % 4] + wfv * buf[i, pl.ds(j * L, L)]
                if tail:
                    xt = buf[i, pl.ds(v - L, L)]
                    xt = jnp.where(iota >= L - tail, xt, 0.0)
                    accs[n_full % 4] = accs[n_full % 4] + wfv * xt
                return tuple(accs)

            a0, a1, a2, a3 = lax.fori_loop(0, CH, row_body, (a0, a1, a2, a3))
            return (a0, a1, a2, a3, gacc, nacc)

        a0, a1, a2, a3, gacc, nacc = lax.fori_loop(
            0, rows // CH, chunk_body, (zv, zv, zv, zv, zv, zv))

        sacc = (a0 + a1) + (a2 + a3)
        numer = -eps * sacc - (CONF - eps) * gacc + c_const * nacc
        stage[...] = numer
        pltpu.sync_copy(stage, out_hbm.at[pl.ds(wid * L, L)])
        stage2[...] = nacc
        pltpu.sync_copy(stage2, out2_hbm.at[pl.ds(wid * L, L)])

    return k


def kernel(prediction, target, mask):
    v = prediction.shape[-1]
    p = prediction.reshape(-1, v)
    t = target.reshape(-1).astype(jnp.int32)
    m = mask.reshape(-1).astype(jnp.int32)
    numer, cnt = _make_sc_loss(p.shape[0], v)(p, t, m)
    return jnp.sum(numer) / jnp.sum(cnt)


# speedup vs baseline: 4.7051x; 2.3060x over previous
import functools
import math

import jax
import jax.numpy as jnp
from jax import lax
from jax.experimental import pallas as pl
from jax.experimental.pallas import tpu as pltpu
from jax.experimental.pallas import tpu_sc as plsc

SMOOTH = 0.1
CONF = 1.0 - SMOOTH
NC, NS, L = 2, 16, 16
NW = NC * NS
R = 200


def _make_sc_loss(batch, v, toks):
    stripe = toks // NW
    kv = stripe // L
    cpb = v // R
    nch = batch * cpb
    assert nch % 2 == 0 and v % R == 0 and toks % NW == 0 and R % 8 == 0
    eps = SMOOTH / (v - 1)
    c_const = CONF * math.log(CONF) + (v - 1) * eps * math.log(eps)

    mesh = plsc.VectorSubcoreMesh(
        core_axis_name="c", subcore_axis_name="s",
        num_cores=NC, num_subcores=NS)

    @functools.partial(
        pl.kernel,
        out_type=(
            jax.ShapeDtypeStruct((NW * L,), jnp.float32),
            jax.ShapeDtypeStruct((NW * L,), jnp.float32),
        ),
        mesh=mesh,
        compiler_params=pltpu.CompilerParams(needs_layout_passes=False),
        scratch_types=[
            pltpu.VMEM((batch * stripe,), jnp.int32),
            pltpu.VMEM((batch * stripe,), jnp.int32),
            pltpu.VMEM((R, stripe), jnp.float32),
            pltpu.VMEM((R, stripe), jnp.float32),
            pltpu.VMEM((L,), jnp.float32),
            pltpu.VMEM((L,), jnp.float32),
            pltpu.SemaphoreType.DMA,
            pltpu.SemaphoreType.DMA,
        ],
    )
    def k(q_hbm, tgt_hbm, msk_hbm, out_hbm, out2_hbm,
          mvec, tvec, buf0, buf1, stage, stage2, sem0, sem1):
        wid = lax.axis_index("s") * NC + lax.axis_index("c")
        col0 = wid * stripe
        iota = lax.iota(jnp.int32, L)

        def bc(x, dtype):
            return lax.broadcast(jnp.asarray(x, dtype), (L,))

        for b in range(batch):
            pltpu.sync_copy(msk_hbm.at[pl.ds(b * toks + col0, stripe)],
                            mvec.at[pl.ds(b * stripe, stripe)])
            pltpu.sync_copy(tgt_hbm.at[pl.ds(b * toks + col0, stripe)],
                            tvec.at[pl.ds(b * stripe, stripe)])

        zv = jnp.zeros((L,), jnp.float32)

        nacc = zv
        for kk in range(batch * kv):
            nacc = nacc + jnp.where(mvec[pl.ds(kk * L, L)] > 0, 1.0, 0.0)

        def start(ci, buf, sem):
            b = ci // cpb
            c = ci - b * cpb
            src = q_hbm.at[pl.ds(b * v + c * R, R), pl.ds(col0, stripe)]
            return pltpu.async_copy(src, buf, sem)

        def process(ci, buf, sem, carry):
            pltpu.make_async_copy(
                q_hbm.at[pl.ds(0, R), pl.ds(col0, stripe)], buf, sem).wait()
            b = ci // cpb
            c = ci - b * cpb
            accs = carry

            def row_body(r, cc):
                cc = list(cc)
                for kk in range(kv):
                    cc[kk] = cc[kk] + buf[r, pl.ds(kk * L, L)]
                return tuple(cc)

            local = lax.fori_loop(0, R, row_body, tuple([zv] * kv))

            out = []
            for kk in range(kv):
                mk = mvec[pl.ds(b * stripe + kk * L, L)]
                wf = jnp.where(mk > 0, 1.0, 0.0)
                tk = tvec[pl.ds(b * stripe + kk * L, L)]
                rowidx = tk - bc(c * R, jnp.int32)
                inb = (rowidx >= 0) & (rowidx < R)
                srow = jnp.where(inb, rowidx, 0)
                val = plsc.load_gather(buf, [srow, kk * L + iota])
                g_add = jnp.where(inb, wf * val, 0.0)
                s_k = accs[kk] + wf * local[kk]
                g_k = accs[kv + kk] + g_add
                out.append((s_k, g_k))
            return tuple(x[0] for x in out) + tuple(x[1] for x in out)

        carry = tuple([zv] * (2 * kv))
        start(0, buf0, sem0)
        start(1, buf1, sem1)

        def pair_body(u, carry):
            ci0 = u * 2
            carry = process(ci0, buf0, sem0, carry)
            start(ci0 + 2, buf0, sem0)
            carry = process(ci0 + 1, buf1, sem1, carry)
            start(ci0 + 3, buf1, sem1)
            return carry

        carry = lax.fori_loop(0, nch // 2 - 1, pair_body, carry)
        carry = process(nch - 2, buf0, sem0, carry)
        carry = process(nch - 1, buf1, sem1, carry)

        stot = carry[0]
        for kk in range(1, kv):
            stot = stot + carry[kk]
        gtot = carry[kv]
        for kk in range(1, kv):
            gtot = gtot + carry[kv + kk]

        numer = -eps * stot - (CONF - eps) * gtot + c_const * nacc
        stage[...] = numer
        pltpu.sync_copy(stage, out_hbm.at[pl.ds(wid * L, L)])
        stage2[...] = nacc
        pltpu.sync_copy(stage2, out2_hbm.at[pl.ds(wid * L, L)])

    return k


def kernel(prediction, target, mask):
    batch, toks, v = prediction.shape
    q = prediction.transpose(0, 2, 1).reshape(batch * v, toks)
    t = target.reshape(-1).astype(jnp.int32)
    m = mask.reshape(-1).astype(jnp.int32)
    numer, cnt = _make_sc_loss(batch, v, toks)(q, t, m)
    return jnp.sum(numer) / jnp.sum(cnt)
